# baseline (speedup 1.0000x reference)
import jax
import jax.numpy as jnp
from jax.experimental import pallas as pl
from jax.experimental.pallas import tpu as pltpu

_B = 16384
_N = 1000
_BLK = 512
_GRID = _B // _BLK
_C = 8
_NCHUNK = _BLK // _C

_P0 = 0.6931502950629682
_P1 = 0.49990933485337247
_P2 = 0.12560248901219037
_P3 = -0.0014526603471430727
_P4 = -0.003951283348970519


def _pass_body(y_ref, lab_ref, out_ref, acc_ref):
    i = pl.program_id(0)

    @pl.when(i == 0)
    def _init():
        for k in range(5):
            acc_ref[k] = 0.0

    col = jax.lax.broadcasted_iota(jnp.int32, (_C, _N), 1)

    def chunk(r, carry):
        a_e2, a_sp, a_lse, a_tv, a_pt = carry
        base = pl.multiple_of(r * jnp.int32(_C), _C)
        y = y_ref[pl.ds(base, _C), :]
        lab = lab_ref[pl.ds(base, _C), :]
        e = jnp.exp(y)
        a_e2 = a_e2 + e * e
        sp = (((_P4 * e + _P3) * e + _P2) * e + _P1) * e + _P0
        a_sp = a_sp + sp
        rs = jnp.sum(e, axis=1, keepdims=True)
        a_lse = a_lse + jnp.log(rs)
        tv = jnp.sum(jnp.where(col == lab, y, 0.0), axis=1, keepdims=True)
        a_tv = a_tv + tv
        a_pt = a_pt + jnp.exp(tv)
        return a_e2, a_sp, a_lse, a_tv, a_pt

    z2 = jnp.zeros((_C, _N), jnp.float32)
    z1 = jnp.zeros((_C, 1), jnp.float32)
    a_e2, a_sp, a_lse, a_tv, a_pt = jax.lax.fori_loop(
        jnp.int32(0), jnp.int32(_NCHUNK), chunk, (z2, z2, z1, z1, z1))

    acc_ref[0] += jnp.sum(a_e2)
    acc_ref[1] += jnp.sum(a_pt)
    acc_ref[2] += jnp.sum(a_tv)
    acc_ref[3] += jnp.sum(a_lse)
    acc_ref[4] += jnp.sum(a_sp)

    @pl.when(i == _GRID - 1)
    def _fin():
        for k in range(5):
            out_ref[k] = acc_ref[k]


def kernel(y_pred, y_true, weights):
    lab = y_true.astype(jnp.int32).reshape(_B, 1)
    sums = pl.pallas_call(
        _pass_body,
        grid=(_GRID,),
        in_specs=[
            pl.BlockSpec((_BLK, _N), lambda i: (i, i * 0)),
            pl.BlockSpec((_BLK, 1), lambda i: (i, i * 0)),
        ],
        out_specs=pl.BlockSpec((5,), lambda i: (i * 0,), memory_space=pltpu.SMEM),
        out_shape=jax.ShapeDtypeStruct((5,), jnp.float32),
        scratch_shapes=[pltpu.SMEM((5,), jnp.float32)],
    )(y_pred, lab)
    s_e2 = sums[0].astype(jnp.float64)
    s_pt = sums[1].astype(jnp.float64)
    s_tval = sums[2].astype(jnp.float64)
    s_lse = sums[3].astype(jnp.float64)
    s_sp = sums[4].astype(jnp.float64)

    w = jax.nn.softplus(weights)
    bn = float(_B * _N)
    mse = (s_e2 - 2.0 * s_pt + float(_B)) / bn
    ce = (s_lse - s_tval) / float(_B)
    mlsm = (s_sp - s_pt) / bn
    return w[0] * mse + w[1] * ce + w[2] * mlsm


# speedup vs baseline: 2.8837x; 2.8837x over previous
import jax
import jax.numpy as jnp
from jax.experimental import pallas as pl
from jax.experimental.pallas import tpu as pltpu

_B = 16384
_N = 1000
_BLK = 512
_GRID = _B // _BLK
_C = 8
_NCHUNK = _BLK // _C

_P0 = 0.6931502950629682
_P1 = 0.49990933485337247
_P2 = 0.12560248901219037
_P3 = -0.0014526603471430727
_P4 = -0.003951283348970519


def _pass_body(y_ref, lab_ref, out_ref, acc_ref):
    i = pl.program_id(0)

    @pl.when(i == 0)
    def _init():
        for k in range(5):
            acc_ref[k] = 0.0

    y = y_ref[...]
    e = jnp.exp(y)
    lab = lab_ref[...]
    col = jax.lax.broadcasted_iota(jnp.int32, (_BLK, _N), 1)
    mask = col == lab

    rowsum = jnp.sum(e, axis=1, keepdims=True)
    s_lse = jnp.sum(jnp.log(rowsum))
    s_e2 = jnp.sum(e * e)
    sp = (((_P4 * e + _P3) * e + _P2) * e + _P1) * e + _P0
    s_sp = jnp.sum(sp)
    tv = jnp.sum(jnp.where(mask, y, 0.0), axis=1, keepdims=True)
    s_tval = jnp.sum(tv)
    s_pt = jnp.sum(jnp.exp(tv))

    acc_ref[0] += s_e2
    acc_ref[1] += s_pt
    acc_ref[2] += s_tval
    acc_ref[3] += s_lse
    acc_ref[4] += s_sp

    @pl.when(i == _GRID - 1)
    def _fin():
        for k in range(5):
            out_ref[k] = acc_ref[k]


def kernel(y_pred, y_true, weights):
    lab = y_true.astype(jnp.int32).reshape(_B, 1)
    sums = pl.pallas_call(
        _pass_body,
        grid=(_GRID,),
        in_specs=[
            pl.BlockSpec((_BLK, _N), lambda i: (i, i * 0)),
            pl.BlockSpec((_BLK, 1), lambda i: (i, i * 0)),
        ],
        out_specs=pl.BlockSpec((5,), lambda i: (i * 0,), memory_space=pltpu.SMEM),
        out_shape=jax.ShapeDtypeStruct((5,), jnp.float32),
        scratch_shapes=[pltpu.SMEM((5,), jnp.float32)],
    )(y_pred, lab)
    s_e2 = sums[0].astype(jnp.float64)
    s_pt = sums[1].astype(jnp.float64)
    s_tval = sums[2].astype(jnp.float64)
    s_lse = sums[3].astype(jnp.float64)
    s_sp = sums[4].astype(jnp.float64)

    w = jax.nn.softplus(weights)
    bn = float(_B * _N)
    mse = (s_e2 - 2.0 * s_pt + float(_B)) / bn
    ce = (s_lse - s_tval) / float(_B)
    mlsm = (s_sp - s_pt) / bn
    return w[0] * mse + w[1] * ce + w[2] * mlsm


# speedup vs baseline: 3.0077x; 1.0430x over previous
import jax
import jax.numpy as jnp
from jax.experimental import pallas as pl
from jax.experimental.pallas import tpu as pltpu

_B = 16384
_N = 1000
_BLK = 1024
_GRID = _B // _BLK
_C = 8
_NCHUNK = _BLK // _C

_P0 = 0.6931502950629682
_P1 = 0.49990933485337247
_P2 = 0.12560248901219037
_P3 = -0.0014526603471430727
_P4 = -0.003951283348970519


def _pass_body(y_ref, lab_ref, out_ref, acc_ref):
    i = pl.program_id(0)

    @pl.when(i == 0)
    def _init():
        for k in range(5):
            acc_ref[k] = 0.0

    y = y_ref[...]
    e = jnp.exp(y)
    lab = lab_ref[...]
    col = jax.lax.broadcasted_iota(jnp.int32, (_BLK, _N), 1)
    mask = col == lab

    rowsum = jnp.sum(e, axis=1, keepdims=True)
    s_lse = jnp.sum(jnp.log(rowsum))
    s_e2 = jnp.sum(e * e)
    sp = (((_P4 * e + _P3) * e + _P2) * e + _P1) * e + _P0
    s_sp = jnp.sum(sp)
    tv = jnp.sum(jnp.where(mask, y, 0.0), axis=1, keepdims=True)
    s_tval = jnp.sum(tv)
    s_pt = jnp.sum(jnp.exp(tv))

    acc_ref[0] += s_e2
    acc_ref[1] += s_pt
    acc_ref[2] += s_tval
    acc_ref[3] += s_lse
    acc_ref[4] += s_sp

    @pl.when(i == _GRID - 1)
    def _fin():
        for k in range(5):
            out_ref[k] = acc_ref[k]


def kernel(y_pred, y_true, weights):
    lab = y_true.astype(jnp.int32).reshape(_B, 1)
    sums = pl.pallas_call(
        _pass_body,
        grid=(_GRID,),
        in_specs=[
            pl.BlockSpec((_BLK, _N), lambda i: (i, i * 0)),
            pl.BlockSpec((_BLK, 1), lambda i: (i, i * 0)),
        ],
        out_specs=pl.BlockSpec((5,), lambda i: (i * 0,), memory_space=pltpu.SMEM),
        out_shape=jax.ShapeDtypeStruct((5,), jnp.float32),
        scratch_shapes=[pltpu.SMEM((5,), jnp.float32)],
    )(y_pred, lab)
    s_e2 = sums[0].astype(jnp.float64)
    s_pt = sums[1].astype(jnp.float64)
    s_tval = sums[2].astype(jnp.float64)
    s_lse = sums[3].astype(jnp.float64)
    s_sp = sums[4].astype(jnp.float64)

    w = jax.nn.softplus(weights)
    bn = float(_B * _N)
    mse = (s_e2 - 2.0 * s_pt + float(_B)) / bn
    ce = (s_lse - s_tval) / float(_B)
    mlsm = (s_sp - s_pt) / bn
    return w[0] * mse + w[1] * ce + w[2] * mlsm


# speedup vs baseline: 3.0630x; 1.0184x over previous
import jax
import jax.numpy as jnp
from jax.experimental import pallas as pl
from jax.experimental.pallas import tpu as pltpu

_B = 16384
_N = 1000
_BLK = 2048
_GRID = _B // _BLK
_C = 8
_NCHUNK = _BLK // _C

_P0 = 0.6931502950629682
_P1 = 0.49990933485337247
_P2 = 0.12560248901219037
_P3 = -0.0014526603471430727
_P4 = -0.003951283348970519


def _pass_body(y_ref, lab_ref, out_ref, acc_ref):
    i = pl.program_id(0)

    @pl.when(i == 0)
    def _init():
        for k in range(5):
            acc_ref[k] = 0.0

    y = y_ref[...]
    e = jnp.exp(y)
    lab = lab_ref[...]
    col = jax.lax.broadcasted_iota(jnp.int32, (_BLK, _N), 1)
    mask = col == lab

    rowsum = jnp.sum(e, axis=1, keepdims=True)
    s_lse = jnp.sum(jnp.log(rowsum))
    s_e2 = jnp.sum(e * e)
    sp = (((_P4 * e + _P3) * e + _P2) * e + _P1) * e + _P0
    s_sp = jnp.sum(sp)
    tv = jnp.sum(jnp.where(mask, y, 0.0), axis=1, keepdims=True)
    s_tval = jnp.sum(tv)
    s_pt = jnp.sum(jnp.exp(tv))

    acc_ref[0] += s_e2
    acc_ref[1] += s_pt
    acc_ref[2] += s_tval
    acc_ref[3] += s_lse
    acc_ref[4] += s_sp

    @pl.when(i == _GRID - 1)
    def _fin():
        for k in range(5):
            out_ref[k] = acc_ref[k]


def kernel(y_pred, y_true, weights):
    lab = y_true.astype(jnp.int32).reshape(_B, 1)
    sums = pl.pallas_call(
        _pass_body,
        grid=(_GRID,),
        in_specs=[
            pl.BlockSpec((_BLK, _N), lambda i: (i, i * 0)),
            pl.BlockSpec((_BLK, 1), lambda i: (i, i * 0)),
        ],
        out_specs=pl.BlockSpec((5,), lambda i: (i * 0,), memory_space=pltpu.SMEM),
        out_shape=jax.ShapeDtypeStruct((5,), jnp.float32),
        scratch_shapes=[pltpu.SMEM((5,), jnp.float32)],
    )(y_pred, lab)
    s_e2 = sums[0].astype(jnp.float64)
    s_pt = sums[1].astype(jnp.float64)
    s_tval = sums[2].astype(jnp.float64)
    s_lse = sums[3].astype(jnp.float64)
    s_sp = sums[4].astype(jnp.float64)

    w = jax.nn.softplus(weights)
    bn = float(_B * _N)
    mse = (s_e2 - 2.0 * s_pt + float(_B)) / bn
    ce = (s_lse - s_tval) / float(_B)
    mlsm = (s_sp - s_pt) / bn
    return w[0] * mse + w[1] * ce + w[2] * mlsm
